# baseline (speedup 1.0000x reference)
import jax
import jax.numpy as jnp
from jax import lax
from jax.experimental import pallas as pl
from jax.experimental.pallas import tpu as pltpu
from jax.experimental.pallas import tpu_sc as plsc

N_NODES = 10000
D_FEAT = 128
N_CLASSES = 40

NP = 10240
RPT = 640
NCH = 128
DH = 64
EB = 128
SB = 16
NSB = 10
NB = SB * NSB
E_PAD = 16 * NB * EB


def _rsqrt16(d):
    i = plsc.bitcast(d, jnp.int32)
    i = jnp.int32(0x5F3759DF) - lax.shift_right_logical(i, 1)
    r = plsc.bitcast(i, jnp.float32)
    for _ in range(3):
        r = r * (1.5 - 0.5 * d * r * r)
    return r


def _sgc_body(xs_hbm, rows_hbm, cols_hbm, out_hbm,
              y_hbm, deg_s, a_s, row_sb, col_sb, gbuf, nbuf, dbuf, ones_v,
              gsem0, gsem1, ssem0, ssem1, dsem):
    gsem = (gsem0, gsem1)
    ssem = (ssem0, ssem1)
    c = lax.axis_index("c")
    s = lax.axis_index("s")
    lo = s * RPT

    for i in range(RPT // 16):
        dbuf[pl.ds(i * 16, 16)] = jnp.zeros((16,), jnp.float32)
    pltpu.sync_copy(dbuf, deg_s.at[pl.ds(lo, RPT)])
    for i in range(EB // 16):
        ones_v[pl.ds(i * 16, 16)] = jnp.ones((16,), jnp.float32)
    plsc.subcore_barrier()

    def _deg_super(sb, carry):
        pltpu.sync_copy(cols_hbm.at[s, pl.ds(sb * SB, SB)], col_sb)
        descs = [pltpu.async_copy(ones_v, deg_s.at[col_sb.at[j]], dsem,
                                  add=True)
                 for j in range(SB)]
        for d in descs:
            d.wait()
        return carry

    lax.fori_loop(0, NSB, _deg_super, 0)
    plsc.subcore_barrier()

    pltpu.sync_copy(deg_s.at[pl.ds(lo, RPT)], dbuf)
    for i in range(RPT // 16):
        dv = dbuf[pl.ds(i * 16, 16)] + 1.0
        dbuf[pl.ds(i * 16, 16)] = _rsqrt16(dv)

    def _scale_nbuf(base, squared):
        def _grp(g, carry):
            dvec = dbuf[pl.ds(base + g * 16, 16)]
            if squared:
                dvec = dvec * dvec
            for r in range(16):
                sc = dvec[r]
                i = g * 16 + r
                for k in range(DH // 16):
                    nbuf[i, pl.ds(k * 16, 16)] = (
                        nbuf[i, pl.ds(k * 16, 16)] * sc)
            return carry
        lax.fori_loop(0, NCH // 16, _grp, 0)

    for g in range(RPT // NCH):
        sub_lo = lo + g * NCH
        pltpu.sync_copy(xs_hbm.at[c, pl.ds(sub_lo, NCH)], nbuf)
        _scale_nbuf(g * NCH, False)
        pltpu.sync_copy(nbuf, y_hbm.at[c, pl.ds(sub_lo, NCH)])
        pltpu.sync_copy(nbuf, a_s.at[pl.ds(sub_lo, NCH)])
    plsc.subcore_barrier()

    def _hop():
        def _super(sb, carry):
            pltpu.sync_copy(rows_hbm.at[s, pl.ds(sb * SB, SB)], row_sb)
            pltpu.sync_copy(cols_hbm.at[s, pl.ds(sb * SB, SB)], col_sb)
            g_descs = [None, None]
            s_descs = [None, None]
            for j in range(SB):
                bi = j & 1
                if j >= 2:
                    s_descs[bi].wait()
                g_descs[bi] = pltpu.async_copy(
                    y_hbm.at[c].at[row_sb.at[j]], gbuf.at[bi], gsem[bi])
                if j >= 1:
                    g_descs[1 - bi].wait()
                    s_descs[1 - bi] = pltpu.async_copy(
                        gbuf.at[1 - bi], a_s.at[col_sb.at[j - 1]],
                        ssem[1 - bi], add=True)
            last = (SB - 1) & 1
            g_descs[last].wait()
            s_descs[1 - last].wait()
            fin = pltpu.async_copy(
                gbuf.at[last], a_s.at[col_sb.at[SB - 1]], ssem[last],
                add=True)
            fin.wait()
            return carry

        lax.fori_loop(0, NSB, _super, 0)

    _hop()
    plsc.subcore_barrier()

    for g in range(RPT // NCH):
        sub_lo = lo + g * NCH
        pltpu.sync_copy(a_s.at[pl.ds(sub_lo, NCH)], nbuf)
        _scale_nbuf(g * NCH, True)
        pltpu.sync_copy(nbuf, y_hbm.at[c, pl.ds(sub_lo, NCH)])
        pltpu.sync_copy(nbuf, a_s.at[pl.ds(sub_lo, NCH)])
    plsc.subcore_barrier()

    _hop()
    plsc.subcore_barrier()

    for g in range(RPT // NCH):
        sub_lo = lo + g * NCH
        pltpu.sync_copy(a_s.at[pl.ds(sub_lo, NCH)], nbuf)
        _scale_nbuf(g * NCH, False)
        pltpu.sync_copy(nbuf, out_hbm.at[c, pl.ds(sub_lo, NCH)])


_sgc_prop = pl.kernel(
    _sgc_body,
    out_type=jax.ShapeDtypeStruct((2, NP, DH), jnp.float32),
    mesh=plsc.VectorSubcoreMesh(core_axis_name="c", subcore_axis_name="s"),
    compiler_params=pltpu.CompilerParams(needs_layout_passes=False,
                                         use_tc_tiling_on_sc=False),
    scratch_types=[
        pltpu.HBM((2, NP, DH), jnp.float32),
        pltpu.VMEM_SHARED((NP,), jnp.float32),
        pltpu.VMEM_SHARED((NP, DH), jnp.float32),
        pltpu.VMEM((SB, EB), jnp.int32),
        pltpu.VMEM((SB, EB), jnp.int32),
        pltpu.VMEM((2, EB, DH), jnp.float32),
        pltpu.VMEM((NCH, DH), jnp.float32),
        pltpu.VMEM((RPT,), jnp.float32),
        pltpu.VMEM((EB,), jnp.float32),
        pltpu.SemaphoreType.DMA,
        pltpu.SemaphoreType.DMA,
        pltpu.SemaphoreType.DMA,
        pltpu.SemaphoreType.DMA,
        pltpu.SemaphoreType.DMA,
    ],
)


_BLK = 1024


def _lin_body(x_ref, wt_ref, b_ref, o_ref):
    l = jnp.dot(x_ref[...], wt_ref[...],
                preferred_element_type=jnp.float32) + b_ref[...]
    m = jnp.max(l, axis=1, keepdims=True)
    e = jnp.exp(l - m)
    ssum = jnp.sum(e, axis=1, keepdims=True)
    o_ref[...] = l - m - jnp.log(ssum)


def _linear_logsoftmax(x2, wt, bvec):
    return pl.pallas_call(
        _lin_body,
        grid=(NP // _BLK,),
        in_specs=[
            pl.BlockSpec((_BLK, D_FEAT), lambda i: (i, 0)),
            pl.BlockSpec((D_FEAT, D_FEAT), lambda i: (0, 0)),
            pl.BlockSpec((1, D_FEAT), lambda i: (0, 0)),
        ],
        out_specs=pl.BlockSpec((_BLK, D_FEAT), lambda i: (i, 0)),
        out_shape=jax.ShapeDtypeStruct((NP, D_FEAT), jnp.float32),
    )(x2, wt, bvec)


def kernel(feature, edge_index, use_feature, W, b):
    f32 = jnp.float32
    x = jnp.where(use_feature != 0, feature.astype(f32),
                  jnp.eye(N_NODES, D_FEAT, dtype=f32))
    x_pad = jnp.zeros((NP, D_FEAT), f32).at[:N_NODES].set(x)
    xs = jnp.stack([x_pad[:, :DH], x_pad[:, DH:]])

    row = edge_index[0].astype(jnp.int32)
    col = edge_index[1].astype(jnp.int32)
    n_edges = row.shape[0]
    rows3 = jnp.zeros((E_PAD,), jnp.int32).at[:n_edges].set(row)
    cols3 = jnp.full((E_PAD,), NP - 1, jnp.int32).at[:n_edges].set(col)
    rows3 = rows3.reshape(16, NB, EB)
    cols3 = cols3.reshape(16, NB, EB)

    h = _sgc_prop(xs, rows3, cols3)
    x2 = jnp.concatenate([h[0], h[1]], axis=1)

    wt = jnp.zeros((D_FEAT, D_FEAT), f32).at[:, :N_CLASSES].set(
        W.astype(f32).T)
    bp = jnp.full((1, D_FEAT), -1e30, f32).at[0, :N_CLASSES].set(
        b.astype(f32))
    out = _linear_logsoftmax(x2, wt, bp)
    return out[:N_NODES, :N_CLASSES]


# speedup vs baseline: 1.7943x; 1.7943x over previous
import jax
import jax.numpy as jnp
from jax import lax
from jax.experimental import pallas as pl
from jax.experimental.pallas import tpu as pltpu
from jax.experimental.pallas import tpu_sc as plsc

N_NODES = 10000
D_FEAT = 128
N_CLASSES = 40

NP = 10240
RPT = 640
NCH = 128
DH = 64
EB = 256
SB = 8
NSB = 10
NB = SB * NSB
E_PAD = 16 * NB * EB


def _rsqrt16(d):
    i = plsc.bitcast(d, jnp.int32)
    i = jnp.int32(0x5F3759DF) - lax.shift_right_logical(i, 1)
    r = plsc.bitcast(i, jnp.float32)
    for _ in range(3):
        r = r * (1.5 - 0.5 * d * r * r)
    return r


def _sgc_body(xs_hbm, rows_hbm, cols_hbm, out_hbm,
              deg_s, y_s, a_s, row_sb, col_sb, gbuf, nbuf, dbuf, ones_v,
              gsem0, gsem1, ssem0, ssem1, dsem):
    gsem = (gsem0, gsem1)
    ssem = (ssem0, ssem1)
    c = lax.axis_index("c")
    s = lax.axis_index("s")
    lo = s * RPT

    for i in range(RPT // 16):
        dbuf[pl.ds(i * 16, 16)] = jnp.zeros((16,), jnp.float32)
    pltpu.sync_copy(dbuf, deg_s.at[pl.ds(lo, RPT)])
    for i in range(EB // 16):
        ones_v[pl.ds(i * 16, 16)] = jnp.ones((16,), jnp.float32)
    plsc.subcore_barrier()

    def _deg_super(sb, carry):
        pltpu.sync_copy(cols_hbm.at[s, pl.ds(sb * SB, SB)], col_sb)
        descs = [pltpu.async_copy(ones_v, deg_s.at[col_sb.at[j]], dsem,
                                  add=True)
                 for j in range(SB)]
        for d in descs:
            d.wait()
        return carry

    lax.fori_loop(0, NSB, _deg_super, 0)
    plsc.subcore_barrier()

    pltpu.sync_copy(deg_s.at[pl.ds(lo, RPT)], dbuf)
    for i in range(RPT // 16):
        dv = dbuf[pl.ds(i * 16, 16)] + 1.0
        dbuf[pl.ds(i * 16, 16)] = _rsqrt16(dv)

    def _scale_nbuf(base, squared):
        def _grp(g, carry):
            dvec = dbuf[pl.ds(base + g * 16, 16)]
            if squared:
                dvec = dvec * dvec
            for r in range(16):
                sc = dvec[r]
                i = g * 16 + r
                for k in range(DH // 16):
                    nbuf[i, pl.ds(k * 16, 16)] = (
                        nbuf[i, pl.ds(k * 16, 16)] * sc)
            return carry
        lax.fori_loop(0, NCH // 16, _grp, 0)

    for g in range(RPT // NCH):
        sub_lo = lo + g * NCH
        pltpu.sync_copy(xs_hbm.at[c, pl.ds(sub_lo, NCH)], nbuf)
        _scale_nbuf(g * NCH, False)
        pltpu.sync_copy(nbuf, y_s.at[pl.ds(sub_lo, NCH)])
        pltpu.sync_copy(nbuf, a_s.at[pl.ds(sub_lo, NCH)])
    plsc.subcore_barrier()

    def _hop():
        def _super(sb, carry):
            pltpu.sync_copy(rows_hbm.at[s, pl.ds(sb * SB, SB)], row_sb)
            pltpu.sync_copy(cols_hbm.at[s, pl.ds(sb * SB, SB)], col_sb)
            g_descs = [None, None]
            s_descs = [None, None]
            for j in range(SB):
                bi = j & 1
                if j >= 2:
                    s_descs[bi].wait()
                g_descs[bi] = pltpu.async_copy(
                    y_s.at[row_sb.at[j]], gbuf.at[bi], gsem[bi])
                if j >= 1:
                    g_descs[1 - bi].wait()
                    s_descs[1 - bi] = pltpu.async_copy(
                        gbuf.at[1 - bi], a_s.at[col_sb.at[j - 1]],
                        ssem[1 - bi], add=True)
            last = (SB - 1) & 1
            g_descs[last].wait()
            s_descs[1 - last].wait()
            fin = pltpu.async_copy(
                gbuf.at[last], a_s.at[col_sb.at[SB - 1]], ssem[last],
                add=True)
            fin.wait()
            return carry

        lax.fori_loop(0, NSB, _super, 0)

    _hop()
    plsc.subcore_barrier()

    for g in range(RPT // NCH):
        sub_lo = lo + g * NCH
        pltpu.sync_copy(a_s.at[pl.ds(sub_lo, NCH)], nbuf)
        _scale_nbuf(g * NCH, True)
        pltpu.sync_copy(nbuf, y_s.at[pl.ds(sub_lo, NCH)])
        pltpu.sync_copy(nbuf, a_s.at[pl.ds(sub_lo, NCH)])
    plsc.subcore_barrier()

    _hop()
    plsc.subcore_barrier()

    for g in range(RPT // NCH):
        sub_lo = lo + g * NCH
        pltpu.sync_copy(a_s.at[pl.ds(sub_lo, NCH)], nbuf)
        _scale_nbuf(g * NCH, False)
        pltpu.sync_copy(nbuf, out_hbm.at[c, pl.ds(sub_lo, NCH)])


_sgc_prop = pl.kernel(
    _sgc_body,
    out_type=jax.ShapeDtypeStruct((2, NP, DH), jnp.float32),
    mesh=plsc.VectorSubcoreMesh(core_axis_name="c", subcore_axis_name="s"),
    compiler_params=pltpu.CompilerParams(needs_layout_passes=False,
                                         use_tc_tiling_on_sc=False),
    scratch_types=[
        pltpu.VMEM_SHARED((NP,), jnp.float32),
        pltpu.VMEM_SHARED((NP, DH), jnp.float32),
        pltpu.VMEM_SHARED((NP, DH), jnp.float32),
        pltpu.VMEM((SB, EB), jnp.int32),
        pltpu.VMEM((SB, EB), jnp.int32),
        pltpu.VMEM((2, EB, DH), jnp.float32),
        pltpu.VMEM((NCH, DH), jnp.float32),
        pltpu.VMEM((RPT,), jnp.float32),
        pltpu.VMEM((EB,), jnp.float32),
        pltpu.SemaphoreType.DMA,
        pltpu.SemaphoreType.DMA,
        pltpu.SemaphoreType.DMA,
        pltpu.SemaphoreType.DMA,
        pltpu.SemaphoreType.DMA,
    ],
)


_BLK = 1024


def _lin_body(x_ref, wt_ref, b_ref, o_ref):
    l = jnp.dot(x_ref[...], wt_ref[...],
                preferred_element_type=jnp.float32) + b_ref[...]
    m = jnp.max(l, axis=1, keepdims=True)
    e = jnp.exp(l - m)
    ssum = jnp.sum(e, axis=1, keepdims=True)
    o_ref[...] = l - m - jnp.log(ssum)


def _linear_logsoftmax(x2, wt, bvec):
    return pl.pallas_call(
        _lin_body,
        grid=(NP // _BLK,),
        in_specs=[
            pl.BlockSpec((_BLK, D_FEAT), lambda i: (i, 0)),
            pl.BlockSpec((D_FEAT, D_FEAT), lambda i: (0, 0)),
            pl.BlockSpec((1, D_FEAT), lambda i: (0, 0)),
        ],
        out_specs=pl.BlockSpec((_BLK, D_FEAT), lambda i: (i, 0)),
        out_shape=jax.ShapeDtypeStruct((NP, D_FEAT), jnp.float32),
    )(x2, wt, bvec)


def kernel(feature, edge_index, use_feature, W, b):
    f32 = jnp.float32
    x = jnp.where(use_feature != 0, feature.astype(f32),
                  jnp.eye(N_NODES, D_FEAT, dtype=f32))
    x_pad = jnp.zeros((NP, D_FEAT), f32).at[:N_NODES].set(x)
    xs = jnp.stack([x_pad[:, :DH], x_pad[:, DH:]])

    row = edge_index[0].astype(jnp.int32)
    col = edge_index[1].astype(jnp.int32)
    n_edges = row.shape[0]
    rows3 = jnp.zeros((E_PAD,), jnp.int32).at[:n_edges].set(row)
    cols3 = jnp.full((E_PAD,), NP - 1, jnp.int32).at[:n_edges].set(col)
    rows3 = rows3.reshape(16, NB, EB)
    cols3 = cols3.reshape(16, NB, EB)

    h = _sgc_prop(xs, rows3, cols3)
    x2 = jnp.concatenate([h[0], h[1]], axis=1)

    wt = jnp.zeros((D_FEAT, D_FEAT), f32).at[:, :N_CLASSES].set(
        W.astype(f32).T)
    bp = jnp.full((1, D_FEAT), -1e30, f32).at[0, :N_CLASSES].set(
        b.astype(f32))
    out = _linear_logsoftmax(x2, wt, bp)
    return out[:N_NODES, :N_CLASSES]


# speedup vs baseline: 2.6023x; 1.4503x over previous
import jax
import jax.numpy as jnp
from jax import lax
from jax.experimental import pallas as pl
from jax.experimental.pallas import tpu as pltpu
from jax.experimental.pallas import tpu_sc as plsc

N_NODES = 10000
D_FEAT = 128
N_CLASSES = 40

NP = 10240
RPT = 640
NCH = 128
DH = 64
EB = 256
SB = 8
NSB = 10
NB = SB * NSB
E_PAD = 16 * NB * EB


def _rsqrt16(d):
    i = plsc.bitcast(d, jnp.int32)
    i = jnp.int32(0x5F3759DF) - lax.shift_right_logical(i, 1)
    r = plsc.bitcast(i, jnp.float32)
    for _ in range(3):
        r = r * (1.5 - 0.5 * d * r * r)
    return r


def _sgc_body(xs_hbm, rows_hbm, cols_hbm, out_hbm,
              deg_s, y_s, a_s, row_sb, col_sb, gbuf, nbuf, nb16, dbuf,
              ones_v, gsem0, gsem1, ssem0, ssem1, dsem):
    gsem = (gsem0, gsem1)
    ssem = (ssem0, ssem1)
    c = lax.axis_index("c")
    s = lax.axis_index("s")
    lo = s * RPT

    for i in range(RPT // 16):
        dbuf[pl.ds(i * 16, 16)] = jnp.zeros((16,), jnp.float32)
    pltpu.sync_copy(dbuf, deg_s.at[pl.ds(lo, RPT)])
    for i in range(EB // 16):
        ones_v[pl.ds(i * 16, 16)] = jnp.ones((16,), jnp.float32)
    plsc.subcore_barrier()

    def _deg_super(sb, carry):
        pltpu.sync_copy(cols_hbm.at[s, pl.ds(sb * SB, SB)], col_sb)
        descs = [pltpu.async_copy(ones_v, deg_s.at[col_sb.at[j]], dsem,
                                  add=True)
                 for j in range(SB)]
        for d in descs:
            d.wait()
        return carry

    lax.fori_loop(0, NSB, _deg_super, 0)
    plsc.subcore_barrier()

    pltpu.sync_copy(deg_s.at[pl.ds(lo, RPT)], dbuf)
    for i in range(RPT // 16):
        dv = dbuf[pl.ds(i * 16, 16)] + 1.0
        dbuf[pl.ds(i * 16, 16)] = _rsqrt16(dv)

    def _scale_chunk(base, squared, src16, dst16):
        def _grp(g, carry):
            dvec = dbuf[pl.ds(base + g * 16, 16)]
            if squared:
                dvec = dvec * dvec
            for r in range(16):
                sc = dvec[r]
                i = g * 16 + r
                for k in range(DH // 32):
                    if src16:
                        v32 = nb16[i, pl.ds(k * 32, 32)]
                        a, b = plsc.unpack(
                            v32, format=plsc.PackFormat.INTERLEAVED)
                    else:
                        a = nbuf[i, pl.ds(k * 32, 16)]
                        b = nbuf[i, pl.ds(k * 32 + 16, 16)]
                    a = a * sc
                    b = b * sc
                    if dst16:
                        nb16[i, pl.ds(k * 32, 32)] = plsc.pack(
                            a, b, format=plsc.PackFormat.INTERLEAVED)
                    else:
                        nbuf[i, pl.ds(k * 32, 16)] = a
                        nbuf[i, pl.ds(k * 32 + 16, 16)] = b
            return carry
        lax.fori_loop(0, NCH // 16, _grp, 0)

    for g in range(RPT // NCH):
        sub_lo = lo + g * NCH
        pltpu.sync_copy(xs_hbm.at[c, pl.ds(sub_lo, NCH)], nbuf)
        _scale_chunk(g * NCH, False, False, True)
        pltpu.sync_copy(nb16, y_s.at[pl.ds(sub_lo, NCH)])
        pltpu.sync_copy(nb16, a_s.at[pl.ds(sub_lo, NCH)])
    plsc.subcore_barrier()

    def _hop():
        def _super(sb, carry):
            pltpu.sync_copy(rows_hbm.at[s, pl.ds(sb * SB, SB)], row_sb)
            pltpu.sync_copy(cols_hbm.at[s, pl.ds(sb * SB, SB)], col_sb)
            g_descs = [None, None]
            s_descs = [None, None]
            for j in range(SB):
                bi = j & 1
                if j >= 2:
                    s_descs[bi].wait()
                g_descs[bi] = pltpu.async_copy(
                    y_s.at[row_sb.at[j]], gbuf.at[bi], gsem[bi])
                if j >= 1:
                    g_descs[1 - bi].wait()
                    s_descs[1 - bi] = pltpu.async_copy(
                        gbuf.at[1 - bi], a_s.at[col_sb.at[j - 1]],
                        ssem[1 - bi], add=True)
            last = (SB - 1) & 1
            g_descs[last].wait()
            s_descs[1 - last].wait()
            fin = pltpu.async_copy(
                gbuf.at[last], a_s.at[col_sb.at[SB - 1]], ssem[last],
                add=True)
            fin.wait()
            return carry

        lax.fori_loop(0, NSB, _super, 0)

    _hop()
    plsc.subcore_barrier()

    for g in range(RPT // NCH):
        sub_lo = lo + g * NCH
        pltpu.sync_copy(a_s.at[pl.ds(sub_lo, NCH)], nb16)
        _scale_chunk(g * NCH, True, True, True)
        pltpu.sync_copy(nb16, y_s.at[pl.ds(sub_lo, NCH)])
        pltpu.sync_copy(nb16, a_s.at[pl.ds(sub_lo, NCH)])
    plsc.subcore_barrier()

    _hop()
    plsc.subcore_barrier()

    for g in range(RPT // NCH):
        sub_lo = lo + g * NCH
        pltpu.sync_copy(a_s.at[pl.ds(sub_lo, NCH)], nb16)
        _scale_chunk(g * NCH, False, True, False)
        pltpu.sync_copy(nbuf, out_hbm.at[c, pl.ds(sub_lo, NCH)])


_sgc_prop = pl.kernel(
    _sgc_body,
    out_type=jax.ShapeDtypeStruct((2, NP, DH), jnp.float32),
    mesh=plsc.VectorSubcoreMesh(core_axis_name="c", subcore_axis_name="s"),
    compiler_params=pltpu.CompilerParams(needs_layout_passes=False,
                                         use_tc_tiling_on_sc=False),
    scratch_types=[
        pltpu.VMEM_SHARED((NP,), jnp.float32),
        pltpu.VMEM_SHARED((NP, DH), jnp.bfloat16),
        pltpu.VMEM_SHARED((NP, DH), jnp.bfloat16),
        pltpu.VMEM((SB, EB), jnp.int32),
        pltpu.VMEM((SB, EB), jnp.int32),
        pltpu.VMEM((2, EB, DH), jnp.bfloat16),
        pltpu.VMEM((NCH, DH), jnp.float32),
        pltpu.VMEM((NCH, DH), jnp.bfloat16),
        pltpu.VMEM((RPT,), jnp.float32),
        pltpu.VMEM((EB,), jnp.float32),
        pltpu.SemaphoreType.DMA,
        pltpu.SemaphoreType.DMA,
        pltpu.SemaphoreType.DMA,
        pltpu.SemaphoreType.DMA,
        pltpu.SemaphoreType.DMA,
    ],
)


_BLK = 1024


def _lin_body(x_ref, wt_ref, b_ref, o_ref):
    l = jnp.dot(x_ref[...], wt_ref[...],
                preferred_element_type=jnp.float32) + b_ref[...]
    m = jnp.max(l, axis=1, keepdims=True)
    e = jnp.exp(l - m)
    ssum = jnp.sum(e, axis=1, keepdims=True)
    o_ref[...] = l - m - jnp.log(ssum)


def _linear_logsoftmax(x2, wt, bvec):
    return pl.pallas_call(
        _lin_body,
        grid=(NP // _BLK,),
        in_specs=[
            pl.BlockSpec((_BLK, D_FEAT), lambda i: (i, 0)),
            pl.BlockSpec((D_FEAT, D_FEAT), lambda i: (0, 0)),
            pl.BlockSpec((1, D_FEAT), lambda i: (0, 0)),
        ],
        out_specs=pl.BlockSpec((_BLK, D_FEAT), lambda i: (i, 0)),
        out_shape=jax.ShapeDtypeStruct((NP, D_FEAT), jnp.float32),
    )(x2, wt, bvec)


def kernel(feature, edge_index, use_feature, W, b):
    f32 = jnp.float32
    x = jnp.where(use_feature != 0, feature.astype(f32),
                  jnp.eye(N_NODES, D_FEAT, dtype=f32))
    x_pad = jnp.zeros((NP, D_FEAT), f32).at[:N_NODES].set(x)
    xs = jnp.stack([x_pad[:, :DH], x_pad[:, DH:]])

    row = edge_index[0].astype(jnp.int32)
    col = edge_index[1].astype(jnp.int32)
    n_edges = row.shape[0]
    rows3 = jnp.zeros((E_PAD,), jnp.int32).at[:n_edges].set(row)
    cols3 = jnp.full((E_PAD,), NP - 1, jnp.int32).at[:n_edges].set(col)
    rows3 = rows3.reshape(16, NB, EB)
    cols3 = cols3.reshape(16, NB, EB)

    h = _sgc_prop(xs, rows3, cols3)
    x2 = jnp.concatenate([h[0], h[1]], axis=1)

    wt = jnp.zeros((D_FEAT, D_FEAT), f32).at[:, :N_CLASSES].set(
        W.astype(f32).T)
    bp = jnp.full((1, D_FEAT), -1e30, f32).at[0, :N_CLASSES].set(
        b.astype(f32))
    out = _linear_logsoftmax(x2, wt, bp)
    return out[:N_NODES, :N_CLASSES]


# speedup vs baseline: 3.2733x; 1.2579x over previous
import jax
import jax.numpy as jnp
from jax import lax
from jax.experimental import pallas as pl
from jax.experimental.pallas import tpu as pltpu
from jax.experimental.pallas import tpu_sc as plsc

N_NODES = 10000
D_FEAT = 128
N_CLASSES = 40

NP = 10240
RPT = 640
NCH = 128
DH = 64
EB = 256
NB = 80
E_PAD = 16 * NB * EB


def _rsqrt16(d):
    i = plsc.bitcast(d, jnp.int32)
    i = jnp.int32(0x5F3759DF) - lax.shift_right_logical(i, 1)
    r = plsc.bitcast(i, jnp.float32)
    for _ in range(3):
        r = r * (1.5 - 0.5 * d * r * r)
    return r


def _sgc_body(xs_hbm, rows_hbm, cols_hbm, out_hbm,
              deg_s, y_s, a_s, row_v, col_v, gbuf, nbuf, nb16, dbuf,
              ones_v, gsem0, gsem1, gsem2, gsem3, ssem0, ssem1, ssem2,
              ssem3, dsem):
    gsem = (gsem0, gsem1, gsem2, gsem3)
    ssem = (ssem0, ssem1, ssem2, ssem3)
    c = lax.axis_index("c")
    s = lax.axis_index("s")
    lo = s * RPT

    pltpu.sync_copy(rows_hbm.at[s], row_v)
    pltpu.sync_copy(cols_hbm.at[s], col_v)

    for i in range(RPT // 16):
        dbuf[pl.ds(i * 16, 16)] = jnp.zeros((16,), jnp.float32)
    pltpu.sync_copy(dbuf, deg_s.at[pl.ds(lo, RPT)])
    for i in range(EB // 16):
        ones_v[pl.ds(i * 16, 16)] = jnp.ones((16,), jnp.float32)
    plsc.subcore_barrier()

    def _deg_grp(t, carry):
        descs = [pltpu.async_copy(ones_v, deg_s.at[col_v.at[8 * t + r]],
                                  dsem, add=True)
                 for r in range(8)]
        for d in descs:
            d.wait()
        return carry

    lax.fori_loop(0, NB // 8, _deg_grp, 0)
    plsc.subcore_barrier()

    pltpu.sync_copy(deg_s.at[pl.ds(lo, RPT)], dbuf)
    for i in range(RPT // 16):
        dv = dbuf[pl.ds(i * 16, 16)] + 1.0
        dbuf[pl.ds(i * 16, 16)] = _rsqrt16(dv)

    def _scale_chunk(base, squared, src16, dst16):
        def _grp(g, carry):
            dvec = dbuf[pl.ds(base + g * 16, 16)]
            if squared:
                dvec = dvec * dvec
            for r in range(16):
                sc = dvec[r]
                i = g * 16 + r
                for k in range(DH // 32):
                    if src16:
                        v32 = nb16[i, pl.ds(k * 32, 32)]
                        a, b = plsc.unpack(
                            v32, format=plsc.PackFormat.INTERLEAVED)
                    else:
                        a = nbuf[i, pl.ds(k * 32, 16)]
                        b = nbuf[i, pl.ds(k * 32 + 16, 16)]
                    a = a * sc
                    b = b * sc
                    if dst16:
                        nb16[i, pl.ds(k * 32, 32)] = plsc.pack(
                            a, b, format=plsc.PackFormat.INTERLEAVED)
                    else:
                        nbuf[i, pl.ds(k * 32, 16)] = a
                        nbuf[i, pl.ds(k * 32 + 16, 16)] = b
            return carry
        lax.fori_loop(0, NCH // 16, _grp, 0)

    for g in range(RPT // NCH):
        sub_lo = lo + g * NCH
        pltpu.sync_copy(xs_hbm.at[c, pl.ds(sub_lo, NCH)], nbuf)
        _scale_chunk(g * NCH, False, False, True)
        pltpu.sync_copy(nb16, y_s.at[pl.ds(sub_lo, NCH)])
        pltpu.sync_copy(nb16, a_s.at[pl.ds(sub_lo, NCH)])
    plsc.subcore_barrier()

    def _wait_gather(j, bi):
        pltpu.make_async_copy(y_s.at[row_v.at[j]], gbuf.at[bi],
                              gsem[bi]).wait()

    def _wait_scatter(j, bi):
        pltpu.make_async_copy(gbuf.at[bi], a_s.at[col_v.at[j]],
                              ssem[bi]).wait()

    def _start_gather(j, bi):
        pltpu.async_copy(y_s.at[row_v.at[j]], gbuf.at[bi], gsem[bi])

    def _start_scatter(j, bi):
        pltpu.async_copy(gbuf.at[bi], a_s.at[col_v.at[j]], ssem[bi],
                         add=True)

    def _hop():
        def _step(t, carry):
            for r in range(4):
                j = 4 * t + r

                @pl.when(t > 0)
                def _():
                    _wait_scatter(j - 4, r)
                _start_gather(j, r)
                rp = (r - 1) % 4
                if r == 0:
                    @pl.when(t > 0)
                    def _():
                        _wait_gather(j - 1, rp)
                        _start_scatter(j - 1, rp)
                else:
                    _wait_gather(j - 1, rp)
                    _start_scatter(j - 1, rp)
            return carry

        lax.fori_loop(0, NB // 4, _step, 0)
        _wait_gather(NB - 1, 3)
        _start_scatter(NB - 1, 3)
        for r in range(4):
            _wait_scatter(NB - 4 + r, r)

    _hop()
    plsc.subcore_barrier()

    for g in range(RPT // NCH):
        sub_lo = lo + g * NCH
        pltpu.sync_copy(a_s.at[pl.ds(sub_lo, NCH)], nb16)
        _scale_chunk(g * NCH, True, True, True)
        pltpu.sync_copy(nb16, y_s.at[pl.ds(sub_lo, NCH)])
        pltpu.sync_copy(nb16, a_s.at[pl.ds(sub_lo, NCH)])
    plsc.subcore_barrier()

    _hop()
    plsc.subcore_barrier()

    for g in range(RPT // NCH):
        sub_lo = lo + g * NCH
        pltpu.sync_copy(a_s.at[pl.ds(sub_lo, NCH)], nb16)
        _scale_chunk(g * NCH, False, True, False)
        pltpu.sync_copy(nbuf, out_hbm.at[c, pl.ds(sub_lo, NCH)])


_sgc_prop = pl.kernel(
    _sgc_body,
    out_type=jax.ShapeDtypeStruct((2, NP, DH), jnp.float32),
    mesh=plsc.VectorSubcoreMesh(core_axis_name="c", subcore_axis_name="s"),
    compiler_params=pltpu.CompilerParams(needs_layout_passes=False,
                                         use_tc_tiling_on_sc=False),
    scratch_types=[
        pltpu.VMEM_SHARED((NP,), jnp.float32),
        pltpu.VMEM_SHARED((NP, DH), jnp.bfloat16),
        pltpu.VMEM_SHARED((NP, DH), jnp.bfloat16),
        pltpu.VMEM((NB, EB), jnp.int32),
        pltpu.VMEM((NB, EB), jnp.int32),
        pltpu.VMEM((4, EB, DH), jnp.bfloat16),
        pltpu.VMEM((NCH, DH), jnp.float32),
        pltpu.VMEM((NCH, DH), jnp.bfloat16),
        pltpu.VMEM((RPT,), jnp.float32),
        pltpu.VMEM((EB,), jnp.float32),
        pltpu.SemaphoreType.DMA,
        pltpu.SemaphoreType.DMA,
        pltpu.SemaphoreType.DMA,
        pltpu.SemaphoreType.DMA,
        pltpu.SemaphoreType.DMA,
        pltpu.SemaphoreType.DMA,
        pltpu.SemaphoreType.DMA,
        pltpu.SemaphoreType.DMA,
        pltpu.SemaphoreType.DMA,
    ],
)


_BLK = 1024


def _lin_body(x_ref, wt_ref, b_ref, o_ref):
    l = jnp.dot(x_ref[...], wt_ref[...],
                preferred_element_type=jnp.float32) + b_ref[...]
    m = jnp.max(l, axis=1, keepdims=True)
    e = jnp.exp(l - m)
    ssum = jnp.sum(e, axis=1, keepdims=True)
    o_ref[...] = l - m - jnp.log(ssum)


def _linear_logsoftmax(x2, wt, bvec):
    return pl.pallas_call(
        _lin_body,
        grid=(NP // _BLK,),
        in_specs=[
            pl.BlockSpec((_BLK, D_FEAT), lambda i: (i, 0)),
            pl.BlockSpec((D_FEAT, D_FEAT), lambda i: (0, 0)),
            pl.BlockSpec((1, D_FEAT), lambda i: (0, 0)),
        ],
        out_specs=pl.BlockSpec((_BLK, D_FEAT), lambda i: (i, 0)),
        out_shape=jax.ShapeDtypeStruct((NP, D_FEAT), jnp.float32),
    )(x2, wt, bvec)


def kernel(feature, edge_index, use_feature, W, b):
    f32 = jnp.float32
    x = jnp.where(use_feature != 0, feature.astype(f32),
                  jnp.eye(N_NODES, D_FEAT, dtype=f32))
    x_pad = jnp.zeros((NP, D_FEAT), f32).at[:N_NODES].set(x)
    xs = jnp.stack([x_pad[:, :DH], x_pad[:, DH:]])

    row = edge_index[0].astype(jnp.int32)
    col = edge_index[1].astype(jnp.int32)
    n_edges = row.shape[0]
    rows3 = jnp.zeros((E_PAD,), jnp.int32).at[:n_edges].set(row)
    cols3 = jnp.full((E_PAD,), NP - 1, jnp.int32).at[:n_edges].set(col)
    rows3 = rows3.reshape(16, NB, EB)
    cols3 = cols3.reshape(16, NB, EB)

    h = _sgc_prop(xs, rows3, cols3)
    x2 = jnp.concatenate([h[0], h[1]], axis=1)

    wt = jnp.zeros((D_FEAT, D_FEAT), f32).at[:, :N_CLASSES].set(
        W.astype(f32).T)
    bp = jnp.full((1, D_FEAT), -1e30, f32).at[0, :N_CLASSES].set(
        b.astype(f32))
    out = _linear_logsoftmax(x2, wt, bp)
    return out[:N_NODES, :N_CLASSES]


# speedup vs baseline: 3.2737x; 1.0001x over previous
import jax
import jax.numpy as jnp
from jax import lax
from jax.experimental import pallas as pl
from jax.experimental.pallas import tpu as pltpu
from jax.experimental.pallas import tpu_sc as plsc

N_NODES = 10000
D_FEAT = 128
N_CLASSES = 40

NP = 10240
RPT = 640
NCH = 128
DH = 64
EB = 256
NB = 80
E_PAD = 16 * NB * EB


def _rsqrt16(d):
    i = plsc.bitcast(d, jnp.int32)
    i = jnp.int32(0x5F3759DF) - lax.shift_right_logical(i, 1)
    r = plsc.bitcast(i, jnp.float32)
    for _ in range(3):
        r = r * (1.5 - 0.5 * d * r * r)
    return r


def _sgc_body(xs_hbm, rows_hbm, cols_hbm, out_hbm,
              deg_s, y_s, a_s, row_v, col_v, gbuf, nbuf, nb16, dbuf,
              ones_v, gsem0, gsem1, gsem2, gsem3, ssem0, ssem1, ssem2,
              ssem3, dsem):
    gsem = (gsem0, gsem1, gsem2, gsem3)
    ssem = (ssem0, ssem1, ssem2, ssem3)
    c = lax.axis_index("c")
    s = lax.axis_index("s")
    lo = s * RPT

    with jax.named_scope("ph_idx"):
        pltpu.sync_copy(rows_hbm.at[s], row_v)
        pltpu.sync_copy(cols_hbm.at[s], col_v)

    for i in range(RPT // 16):
        dbuf[pl.ds(i * 16, 16)] = jnp.zeros((16,), jnp.float32)
    pltpu.sync_copy(dbuf, deg_s.at[pl.ds(lo, RPT)])
    for i in range(EB // 16):
        ones_v[pl.ds(i * 16, 16)] = jnp.ones((16,), jnp.float32)
    plsc.subcore_barrier()

    def _deg_grp(t, carry):
        descs = [pltpu.async_copy(ones_v, deg_s.at[col_v.at[8 * t + r]],
                                  dsem, add=True)
                 for r in range(8)]
        for d in descs:
            d.wait()
        return carry

    with jax.named_scope("ph_deg"):
        lax.fori_loop(0, NB // 8, _deg_grp, 0)
    plsc.subcore_barrier()

    with jax.named_scope("ph_dinv"):
        pltpu.sync_copy(deg_s.at[pl.ds(lo, RPT)], dbuf)
        for i in range(RPT // 16):
            dv = dbuf[pl.ds(i * 16, 16)] + 1.0
            dbuf[pl.ds(i * 16, 16)] = _rsqrt16(dv)

    def _scale_chunk(base, squared, src16, dst16):
        def _grp(g, carry):
            dvec = dbuf[pl.ds(base + g * 16, 16)]
            if squared:
                dvec = dvec * dvec
            for r in range(16):
                sc = dvec[r]
                i = g * 16 + r
                for k in range(DH // 32):
                    if src16:
                        v32 = nb16[i, pl.ds(k * 32, 32)]
                        a, b = plsc.unpack(
                            v32, format=plsc.PackFormat.INTERLEAVED)
                    else:
                        a = nbuf[i, pl.ds(k * 32, 16)]
                        b = nbuf[i, pl.ds(k * 32 + 16, 16)]
                    a = a * sc
                    b = b * sc
                    if dst16:
                        nb16[i, pl.ds(k * 32, 32)] = plsc.pack(
                            a, b, format=plsc.PackFormat.INTERLEAVED)
                    else:
                        nbuf[i, pl.ds(k * 32, 16)] = a
                        nbuf[i, pl.ds(k * 32 + 16, 16)] = b
            return carry
        lax.fori_loop(0, NCH // 16, _grp, 0)

    with jax.named_scope("ph_z0"):
      for g in range(RPT // NCH):
        sub_lo = lo + g * NCH
        pltpu.sync_copy(xs_hbm.at[c, pl.ds(sub_lo, NCH)], nbuf)
        _scale_chunk(g * NCH, False, False, True)
        pltpu.sync_copy(nb16, y_s.at[pl.ds(sub_lo, NCH)])
        pltpu.sync_copy(nb16, a_s.at[pl.ds(sub_lo, NCH)])
    plsc.subcore_barrier()

    def _wait_gather(j, bi):
        pltpu.make_async_copy(y_s.at[row_v.at[j]], gbuf.at[bi],
                              gsem[bi]).wait()

    def _wait_scatter(j, bi):
        pltpu.make_async_copy(gbuf.at[bi], a_s.at[col_v.at[j]],
                              ssem[bi]).wait()

    def _start_gather(j, bi):
        pltpu.async_copy(y_s.at[row_v.at[j]], gbuf.at[bi], gsem[bi])

    def _start_scatter(j, bi):
        pltpu.async_copy(gbuf.at[bi], a_s.at[col_v.at[j]], ssem[bi],
                         add=True)

    def _hop():
        def _step(t, carry):
            for r in range(4):
                j = 4 * t + r

                @pl.when(t > 0)
                def _():
                    _wait_scatter(j - 4, r)
                _start_gather(j, r)
                rp = (r - 1) % 4
                if r == 0:
                    @pl.when(t > 0)
                    def _():
                        _wait_gather(j - 1, rp)
                        _start_scatter(j - 1, rp)
                else:
                    _wait_gather(j - 1, rp)
                    _start_scatter(j - 1, rp)
            return carry

        lax.fori_loop(0, NB // 4, _step, 0)
        _wait_gather(NB - 1, 3)
        _start_scatter(NB - 1, 3)
        for r in range(4):
            _wait_scatter(NB - 4 + r, r)

    with jax.named_scope("ph_hop1"):
        _hop()
    plsc.subcore_barrier()

    with jax.named_scope("ph_z1"):
      for g in range(RPT // NCH):
        sub_lo = lo + g * NCH
        pltpu.sync_copy(a_s.at[pl.ds(sub_lo, NCH)], nb16)
        _scale_chunk(g * NCH, True, True, True)
        pltpu.sync_copy(nb16, y_s.at[pl.ds(sub_lo, NCH)])
        pltpu.sync_copy(nb16, a_s.at[pl.ds(sub_lo, NCH)])
    plsc.subcore_barrier()

    with jax.named_scope("ph_hop2"):
        _hop()
    plsc.subcore_barrier()

    with jax.named_scope("ph_fin"):
      for g in range(RPT // NCH):
        sub_lo = lo + g * NCH
        pltpu.sync_copy(a_s.at[pl.ds(sub_lo, NCH)], nb16)
        _scale_chunk(g * NCH, False, True, False)
        pltpu.sync_copy(nbuf, out_hbm.at[c, pl.ds(sub_lo, NCH)])


_sgc_prop = pl.kernel(
    _sgc_body,
    out_type=jax.ShapeDtypeStruct((2, NP, DH), jnp.float32),
    mesh=plsc.VectorSubcoreMesh(core_axis_name="c", subcore_axis_name="s"),
    compiler_params=pltpu.CompilerParams(needs_layout_passes=False,
                                         use_tc_tiling_on_sc=False),
    scratch_types=[
        pltpu.VMEM_SHARED((NP,), jnp.float32),
        pltpu.VMEM_SHARED((NP, DH), jnp.bfloat16),
        pltpu.VMEM_SHARED((NP, DH), jnp.bfloat16),
        pltpu.VMEM((NB, EB), jnp.int32),
        pltpu.VMEM((NB, EB), jnp.int32),
        pltpu.VMEM((4, EB, DH), jnp.bfloat16),
        pltpu.VMEM((NCH, DH), jnp.float32),
        pltpu.VMEM((NCH, DH), jnp.bfloat16),
        pltpu.VMEM((RPT,), jnp.float32),
        pltpu.VMEM((EB,), jnp.float32),
        pltpu.SemaphoreType.DMA,
        pltpu.SemaphoreType.DMA,
        pltpu.SemaphoreType.DMA,
        pltpu.SemaphoreType.DMA,
        pltpu.SemaphoreType.DMA,
        pltpu.SemaphoreType.DMA,
        pltpu.SemaphoreType.DMA,
        pltpu.SemaphoreType.DMA,
        pltpu.SemaphoreType.DMA,
    ],
)


_BLK = 1024


def _lin_body(x_ref, wt_ref, b_ref, o_ref):
    l = jnp.dot(x_ref[...], wt_ref[...],
                preferred_element_type=jnp.float32) + b_ref[...]
    m = jnp.max(l, axis=1, keepdims=True)
    e = jnp.exp(l - m)
    ssum = jnp.sum(e, axis=1, keepdims=True)
    o_ref[...] = l - m - jnp.log(ssum)


def _linear_logsoftmax(x2, wt, bvec):
    return pl.pallas_call(
        _lin_body,
        grid=(NP // _BLK,),
        in_specs=[
            pl.BlockSpec((_BLK, D_FEAT), lambda i: (i, 0)),
            pl.BlockSpec((D_FEAT, D_FEAT), lambda i: (0, 0)),
            pl.BlockSpec((1, D_FEAT), lambda i: (0, 0)),
        ],
        out_specs=pl.BlockSpec((_BLK, D_FEAT), lambda i: (i, 0)),
        out_shape=jax.ShapeDtypeStruct((NP, D_FEAT), jnp.float32),
    )(x2, wt, bvec)


def kernel(feature, edge_index, use_feature, W, b):
    f32 = jnp.float32
    x = jnp.where(use_feature != 0, feature.astype(f32),
                  jnp.eye(N_NODES, D_FEAT, dtype=f32))
    x_pad = jnp.zeros((NP, D_FEAT), f32).at[:N_NODES].set(x)
    xs = jnp.stack([x_pad[:, :DH], x_pad[:, DH:]])

    row = edge_index[0].astype(jnp.int32)
    col = edge_index[1].astype(jnp.int32)
    n_edges = row.shape[0]
    rows3 = jnp.zeros((E_PAD,), jnp.int32).at[:n_edges].set(row)
    cols3 = jnp.full((E_PAD,), NP - 1, jnp.int32).at[:n_edges].set(col)
    rows3 = rows3.reshape(16, NB, EB)
    cols3 = cols3.reshape(16, NB, EB)

    h = _sgc_prop(xs, rows3, cols3)
    x2 = jnp.concatenate([h[0], h[1]], axis=1)

    wt = jnp.zeros((D_FEAT, D_FEAT), f32).at[:, :N_CLASSES].set(
        W.astype(f32).T)
    bp = jnp.full((1, D_FEAT), -1e30, f32).at[0, :N_CLASSES].set(
        b.astype(f32))
    out = _linear_logsoftmax(x2, wt, bp)
    return out[:N_NODES, :N_CLASSES]


# speedup vs baseline: 3.8912x; 1.1886x over previous
import jax
import jax.numpy as jnp
from jax import lax
from jax.experimental import pallas as pl
from jax.experimental.pallas import tpu as pltpu
from jax.experimental.pallas import tpu_sc as plsc

N_NODES = 10000
D_FEAT = 128
N_CLASSES = 40
N_EDGE = 320000

NP = 10240
RPT = 640
NCH = 128
DH = 64
EB = 256
NBT = N_EDGE // EB
NB = NBT // 16
EXTRA = NBT - 16 * NB
NBM = (NB // 4) * 4


def _rsqrt16(d):
    i = plsc.bitcast(d, jnp.int32)
    i = jnp.int32(0x5F3759DF) - lax.shift_right_logical(i, 1)
    r = plsc.bitcast(i, jnp.float32)
    for _ in range(3):
        r = r * (1.5 - 0.5 * d * r * r)
    return r


def _sgc_body(x_hbm, ei_hbm, out_hbm,
              deg_s, y_s, a_s, row_v, col_v, gbuf, nbuf, nb16, dbuf,
              ones_v, gsem0, gsem1, gsem2, gsem3, ssem0, ssem1, ssem2,
              ssem3, dsem):
    gsem = (gsem0, gsem1, gsem2, gsem3)
    ssem = (ssem0, ssem1, ssem2, ssem3)
    c = lax.axis_index("c")
    s = lax.axis_index("s")
    lo = s * RPT

    idx_descs = [
        pltpu.async_copy(ei_hbm.at[0, pl.ds(s * NB, NB)],
                         row_v.at[pl.ds(0, NB)], gsem0),
        pltpu.async_copy(ei_hbm.at[1, pl.ds(s * NB, NB)],
                         col_v.at[pl.ds(0, NB)], gsem1),
    ]

    @pl.when(s < EXTRA)
    def _():
        pltpu.async_copy(ei_hbm.at[0, pl.ds(16 * NB + s, 1)],
                         row_v.at[pl.ds(NB, 1)], gsem2)
        pltpu.async_copy(ei_hbm.at[1, pl.ds(16 * NB + s, 1)],
                         col_v.at[pl.ds(NB, 1)], gsem3)

    for i in range(RPT // 16):
        dbuf[pl.ds(i * 16, 16)] = jnp.zeros((16,), jnp.float32)
    pltpu.sync_copy(dbuf, deg_s.at[pl.ds(lo, RPT)])
    for i in range(EB // 16):
        ones_v[pl.ds(i * 16, 16)] = jnp.ones((16,), jnp.float32)
    for d in idx_descs:
        d.wait()

    @pl.when(s < EXTRA)
    def _():
        pltpu.make_async_copy(ei_hbm.at[0, pl.ds(16 * NB + s, 1)],
                              row_v.at[pl.ds(NB, 1)], gsem2).wait()
        pltpu.make_async_copy(ei_hbm.at[1, pl.ds(16 * NB + s, 1)],
                              col_v.at[pl.ds(NB, 1)], gsem3).wait()
    plsc.subcore_barrier()

    def _col_at(j):
        return col_v.at[j]

    def _row_at(j):
        return row_v.at[j]

    def _deg_grp(t, carry):
        descs = [pltpu.async_copy(ones_v, deg_s.at[_col_at(6 * t + r)],
                                  dsem, add=True)
                 for r in range(6)]
        for d in descs:
            d.wait()
        return carry

    lax.fori_loop(0, NB // 6, _deg_grp, 0)

    @pl.when(s < EXTRA)
    def _():
        pltpu.sync_copy(ones_v, deg_s.at[_col_at(NB)], add=True)
    plsc.subcore_barrier()

    pltpu.sync_copy(deg_s.at[pl.ds(lo, RPT)], dbuf)
    for i in range(RPT // 16):
        dv = dbuf[pl.ds(i * 16, 16)] + 1.0
        dbuf[pl.ds(i * 16, 16)] = _rsqrt16(dv)

    def _scale_chunk(base, squared, src16, dst16=True):
        def _grp(g, carry):
            dvec = dbuf[pl.ds(base + g * 16, 16)]
            if squared:
                dvec = dvec * dvec
            for r in range(16):
                sc = dvec[r]
                i = g * 16 + r
                for k in range(DH // 32):
                    if src16:
                        v32 = nb16[i, pl.ds(k * 32, 32)]
                        a, b = plsc.unpack(
                            v32, format=plsc.PackFormat.INTERLEAVED)
                    else:
                        a = nbuf[i, pl.ds(k * 32, 16)]
                        b = nbuf[i, pl.ds(k * 32 + 16, 16)]
                    a = a * sc
                    b = b * sc
                    if dst16:
                        nb16[i, pl.ds(k * 32, 32)] = plsc.pack(
                            a, b, format=plsc.PackFormat.INTERLEAVED)
                    else:
                        nbuf[i, pl.ds(k * 32, 16)] = a
                        nbuf[i, pl.ds(k * 32 + 16, 16)] = b
            return carry
        lax.fori_loop(0, NCH // 16, _grp, 0)

    for g in range(RPT // NCH):
        sub_lo = lo + g * NCH
        pltpu.sync_copy(x_hbm.at[c, pl.ds(sub_lo, NCH)], nbuf)
        _scale_chunk(g * NCH, False, False)
        pltpu.sync_copy(nb16, y_s.at[pl.ds(sub_lo, NCH)])
        pltpu.sync_copy(nb16, a_s.at[pl.ds(sub_lo, NCH)])
    plsc.subcore_barrier()

    def _wait_gather(j, bi):
        pltpu.make_async_copy(y_s.at[_row_at(j)], gbuf.at[bi],
                              gsem[bi]).wait()

    def _wait_scatter(j, bi):
        pltpu.make_async_copy(gbuf.at[bi], a_s.at[_col_at(j)],
                              ssem[bi]).wait()

    def _start_gather(j, bi):
        pltpu.async_copy(y_s.at[_row_at(j)], gbuf.at[bi], gsem[bi])

    def _start_scatter(j, bi):
        pltpu.async_copy(gbuf.at[bi], a_s.at[_col_at(j)], ssem[bi],
                         add=True)

    def _hop():
        def _step(t, carry):
            for r in range(4):
                j = 4 * t + r

                @pl.when(t > 0)
                def _():
                    _wait_scatter(j - 4, r)
                _start_gather(j, r)
                rp = (r - 1) % 4
                if r == 0:
                    @pl.when(t > 0)
                    def _():
                        _wait_gather(j - 1, rp)
                        _start_scatter(j - 1, rp)
                else:
                    _wait_gather(j - 1, rp)
                    _start_scatter(j - 1, rp)
            return carry

        lax.fori_loop(0, NBM // 4, _step, 0)
        for j in range(NBM, NB):
            bi = j - NBM
            _wait_scatter(j - 4, bi)
            _start_gather(j, bi)
            _wait_gather(j - 1, (j - 1) % 4)
            _start_scatter(j - 1, (j - 1) % 4)
        _wait_gather(NB - 1, (NB - 1) % 4)
        _start_scatter(NB - 1, (NB - 1) % 4)
        for r in range(4):
            _wait_scatter(NB - 4 + r, (NB - 4 + r) % 4)

        @pl.when(s < EXTRA)
        def _():
            pltpu.sync_copy(y_s.at[_row_at(NB)], gbuf.at[0])
            pltpu.sync_copy(gbuf.at[0], a_s.at[_col_at(NB)], add=True)

    _hop()
    plsc.subcore_barrier()

    for g in range(RPT // NCH):
        sub_lo = lo + g * NCH
        pltpu.sync_copy(a_s.at[pl.ds(sub_lo, NCH)], nb16)
        _scale_chunk(g * NCH, True, True)
        pltpu.sync_copy(nb16, y_s.at[pl.ds(sub_lo, NCH)])
        pltpu.sync_copy(nb16, a_s.at[pl.ds(sub_lo, NCH)])
    plsc.subcore_barrier()

    _hop()
    plsc.subcore_barrier()

    for g in range(RPT // NCH):
        sub_lo = lo + g * NCH
        pltpu.sync_copy(a_s.at[pl.ds(sub_lo, NCH)], nb16)
        _scale_chunk(g * NCH, False, True, dst16=False)
        pltpu.sync_copy(nbuf, out_hbm.at[c, pl.ds(sub_lo, NCH)])


_sgc_prop = pl.kernel(
    _sgc_body,
    out_type=jax.ShapeDtypeStruct((2, NP, DH), jnp.float32),
    mesh=plsc.VectorSubcoreMesh(core_axis_name="c", subcore_axis_name="s"),
    compiler_params=pltpu.CompilerParams(needs_layout_passes=False,
                                         use_tc_tiling_on_sc=False),
    scratch_types=[
        pltpu.VMEM_SHARED((NP,), jnp.float32),
        pltpu.VMEM_SHARED((NP, DH), jnp.bfloat16),
        pltpu.VMEM_SHARED((NP, DH), jnp.bfloat16),
        pltpu.VMEM((NB + 1, EB), jnp.int32),
        pltpu.VMEM((NB + 1, EB), jnp.int32),
        pltpu.VMEM((4, EB, DH), jnp.bfloat16),
        pltpu.VMEM((NCH, DH), jnp.float32),
        pltpu.VMEM((NCH, DH), jnp.bfloat16),
        pltpu.VMEM((RPT,), jnp.float32),
        pltpu.VMEM((EB,), jnp.float32),
        pltpu.SemaphoreType.DMA,
        pltpu.SemaphoreType.DMA,
        pltpu.SemaphoreType.DMA,
        pltpu.SemaphoreType.DMA,
        pltpu.SemaphoreType.DMA,
        pltpu.SemaphoreType.DMA,
        pltpu.SemaphoreType.DMA,
        pltpu.SemaphoreType.DMA,
        pltpu.SemaphoreType.DMA,
    ],
)


_BLK = 1024


def _lin_body(x0_ref, x1_ref, wt0_ref, wt1_ref, b_ref, o_ref):
    l = (jnp.dot(x0_ref[0], wt0_ref[...],
                 preferred_element_type=jnp.float32)
         + jnp.dot(x1_ref[0], wt1_ref[...],
                   preferred_element_type=jnp.float32)
         + b_ref[...])
    m = jnp.max(l, axis=1, keepdims=True)
    e = jnp.exp(l - m)
    ssum = jnp.sum(e, axis=1, keepdims=True)
    o_ref[...] = (l - m - jnp.log(ssum))[:, :N_CLASSES]


def _linear_logsoftmax(h, wt0, wt1, bvec):
    return pl.pallas_call(
        _lin_body,
        grid=(NP // _BLK,),
        in_specs=[
            pl.BlockSpec((1, _BLK, DH), lambda i: (0, i, 0)),
            pl.BlockSpec((1, _BLK, DH), lambda i: (1, i, 0)),
            pl.BlockSpec((DH, D_FEAT), lambda i: (0, 0)),
            pl.BlockSpec((DH, D_FEAT), lambda i: (0, 0)),
            pl.BlockSpec((1, D_FEAT), lambda i: (0, 0)),
        ],
        out_specs=pl.BlockSpec((_BLK, N_CLASSES), lambda i: (i, 0)),
        out_shape=jax.ShapeDtypeStruct((NP, N_CLASSES), jnp.float32),
    )(h, h, wt0, wt1, bvec)


def kernel(feature, edge_index, use_feature, W, b):
    f32 = jnp.float32
    x = jnp.where(use_feature != 0, feature.astype(f32),
                  jnp.eye(N_NODES, D_FEAT, dtype=f32))
    x_pad = jnp.zeros((NP, D_FEAT), f32).at[:N_NODES].set(x)
    xs = jnp.stack([x_pad[:, :DH], x_pad[:, DH:]])
    ei = edge_index.astype(jnp.int32).reshape(2, NBT, EB)

    h = _sgc_prop(xs, ei)

    wt = W.astype(f32).T
    wt0 = jnp.zeros((DH, D_FEAT), f32).at[:, :N_CLASSES].set(wt[:DH])
    wt1 = jnp.zeros((DH, D_FEAT), f32).at[:, :N_CLASSES].set(wt[DH:])
    bp = jnp.full((1, D_FEAT), -1e30, f32).at[0, :N_CLASSES].set(
        b.astype(f32))
    out = _linear_logsoftmax(h, wt0, wt1, bp)
    return out[:N_NODES]
